# baseline (speedup 1.0000x reference)
import dataclasses
import jax
import jax.numpy as jnp
from jax import lax
from jax.experimental import pallas as pl
from jax.experimental.pallas import tpu as pltpu
from jax.experimental.pallas import tpu_sc as plsc

_NPIX = float(512 * 512)
_NW = 32
_SPAN = (512 * 512) // _NW


def _sc_hist(o_hbm, t_hbm, out_hbm, b0, b1, b2, b3, tb, bins, sem):
    c = lax.axis_index("c")
    s = lax.axis_index("s")
    w = s * 2 + c
    base = w * _SPAN
    zeros16 = jnp.zeros((16,), jnp.int32)
    ones16 = jnp.ones((16,), jnp.int32)
    lanes = jax.lax.iota(jnp.int32, 16)

    @pl.loop(0, 8)
    def _batch(bi):
        @pl.loop(0, 16)
        def _z(j):
            bins.at[j][...] = zeros16

        pltpu.async_copy(o_hbm.at[bi, 0, pl.ds(base, _SPAN)], b0, sem).wait()
        pltpu.async_copy(o_hbm.at[bi, 1, pl.ds(base, _SPAN)], b1, sem).wait()
        pltpu.async_copy(o_hbm.at[bi, 2, pl.ds(base, _SPAN)], b2, sem).wait()
        pltpu.async_copy(o_hbm.at[bi, 3, pl.ds(base, _SPAN)], b3, sem).wait()
        pltpu.async_copy(t_hbm.at[bi, pl.ds(base, _SPAN)], tb, sem).wait()

        @pl.loop(0, _SPAN, step=16)
        def _vec(i):
            o0 = b0.at[pl.ds(i, 16)][...]
            o1 = b1.at[pl.ds(i, 16)][...]
            o2 = b2.at[pl.ds(i, 16)][...]
            o3 = b3.at[pl.ds(i, 16)][...]
            tv = tb.at[pl.ds(i, 16)][...]
            gt1 = o1 > o0
            gt3 = o3 > o2
            gtb = jnp.maximum(o2, o3) > jnp.maximum(o0, o1)
            idx = jnp.where(gtb,
                            jnp.where(gt3, jnp.int32(3), jnp.int32(2)),
                            jnp.where(gt1, jnp.int32(1), jnp.int32(0)))
            v = (idx << 2) | tv
            plsc.addupdate_scatter(bins, [lanes, v], ones16)

        pltpu.async_copy(bins, out_hbm.at[w, bi], sem).wait()


def _make_sc_kernel():
    cp = pltpu.CompilerParams()
    if "needs_layout_passes" in pltpu.CompilerParams.__dataclass_fields__:
        cp = dataclasses.replace(cp, needs_layout_passes=False)
    mesh = plsc.VectorSubcoreMesh(core_axis_name="c", subcore_axis_name="s")
    return pl.kernel(
        _sc_hist,
        out_type=jax.ShapeDtypeStruct((_NW, 8, 16, 16), jnp.int32),
        mesh=mesh,
        scratch_types=[
            pltpu.VMEM((_SPAN,), jnp.float32),
            pltpu.VMEM((_SPAN,), jnp.float32),
            pltpu.VMEM((_SPAN,), jnp.float32),
            pltpu.VMEM((_SPAN,), jnp.float32),
            pltpu.VMEM((_SPAN,), jnp.int32),
            pltpu.VMEM((16, 16), jnp.int32),
            pltpu.SemaphoreType.DMA,
        ],
        compiler_params=cp,
    )


def _combine_body(bins_ref, score_ref):
    j = jnp.sum(bins_ref[...], axis=(0, 2)).astype(jnp.float32)
    inter = jnp.stack([j[:, 0], j[:, 5], j[:, 10], j[:, 15]], axis=1)
    p = [j[:, 4 * c] + j[:, 4 * c + 1] + j[:, 4 * c + 2] + j[:, 4 * c + 3]
         for c in range(4)]
    t = [j[:, c] + j[:, 4 + c] + j[:, 8 + c] + j[:, 12 + c]
         for c in range(4)]
    card = (jnp.stack(p, axis=1) + jnp.stack(t, axis=1))
    score_ref[...] = jnp.mean(
        2.0 * inter / jnp.maximum(card, 1.0), axis=0, keepdims=True)


def kernel(output, target):
    o2 = output.reshape(8, 4, 512 * 512)
    t2 = target.reshape(8, 512 * 512)
    bins = _make_sc_kernel()(o2, t2)
    score = pl.pallas_call(
        _combine_body,
        grid=(1,),
        in_specs=[pl.BlockSpec((_NW, 8, 16, 16), lambda i: (0, 0, 0, 0))],
        out_specs=pl.BlockSpec((1, 4), lambda i: (0, 0)),
        out_shape=jax.ShapeDtypeStruct((1, 4), jnp.float32),
    )(bins)
    return score[0]


# speedup vs baseline: 1.1329x; 1.1329x over previous
import dataclasses
import jax
import jax.numpy as jnp
from jax import lax
from jax.experimental import pallas as pl
from jax.experimental.pallas import tpu as pltpu
from jax.experimental.pallas import tpu_sc as plsc

_NPIX = float(512 * 512)
_NW = 32
_SPAN = (512 * 512) // _NW


def _sc_hist(o_hbm, t_hbm, out_hbm, b0, b1, b2, b3, tb, bins, sem):
    c = lax.axis_index("c")
    s = lax.axis_index("s")
    w = s * 2 + c
    base = w * _SPAN
    zeros16 = jnp.zeros((16,), jnp.int32)
    ones16 = jnp.ones((16,), jnp.int32)
    lanes = jax.lax.iota(jnp.int32, 16)

    @pl.loop(0, 8)
    def _batch(bi):
        @pl.loop(0, 16)
        def _z(j):
            bins.at[j][...] = zeros16

        cp0 = pltpu.async_copy(o_hbm.at[bi, 0, pl.ds(base, _SPAN)], b0, sem)
        cp1 = pltpu.async_copy(o_hbm.at[bi, 1, pl.ds(base, _SPAN)], b1, sem)
        cp2 = pltpu.async_copy(o_hbm.at[bi, 2, pl.ds(base, _SPAN)], b2, sem)
        cp3 = pltpu.async_copy(o_hbm.at[bi, 3, pl.ds(base, _SPAN)], b3, sem)
        cp4 = pltpu.async_copy(t_hbm.at[bi, pl.ds(base, _SPAN)], tb, sem)
        cp0.wait()
        cp1.wait()
        cp2.wait()
        cp3.wait()
        cp4.wait()

        @pl.loop(0, _SPAN, step=64)
        def _vec(i):
            for u in range(4):
                sl = pl.ds(i + u * 16, 16)
                o0 = b0.at[sl][...]
                o1 = b1.at[sl][...]
                o2 = b2.at[sl][...]
                o3 = b3.at[sl][...]
                tv = tb.at[sl][...]
                gt1 = o1 > o0
                gt3 = o3 > o2
                gtb = jnp.maximum(o2, o3) > jnp.maximum(o0, o1)
                idx = jnp.where(gtb,
                                jnp.where(gt3, jnp.int32(3), jnp.int32(2)),
                                jnp.where(gt1, jnp.int32(1), jnp.int32(0)))
                v = (idx << 2) | tv
                plsc.addupdate_scatter(bins, [lanes, v], ones16)

        pltpu.async_copy(bins, out_hbm.at[w, bi], sem).wait()


def _make_sc_kernel():
    cp = pltpu.CompilerParams()
    if "needs_layout_passes" in pltpu.CompilerParams.__dataclass_fields__:
        cp = dataclasses.replace(cp, needs_layout_passes=False)
    mesh = plsc.VectorSubcoreMesh(core_axis_name="c", subcore_axis_name="s")
    return pl.kernel(
        _sc_hist,
        out_type=jax.ShapeDtypeStruct((_NW, 8, 16, 16), jnp.int32),
        mesh=mesh,
        scratch_types=[
            pltpu.VMEM((_SPAN,), jnp.float32),
            pltpu.VMEM((_SPAN,), jnp.float32),
            pltpu.VMEM((_SPAN,), jnp.float32),
            pltpu.VMEM((_SPAN,), jnp.float32),
            pltpu.VMEM((_SPAN,), jnp.int32),
            pltpu.VMEM((16, 16), jnp.int32),
            pltpu.SemaphoreType.DMA,
        ],
        compiler_params=cp,
    )


def _combine_body(bins_ref, score_ref):
    j = jnp.sum(bins_ref[...], axis=(0, 2)).astype(jnp.float32)
    inter = jnp.stack([j[:, 0], j[:, 5], j[:, 10], j[:, 15]], axis=1)
    p = [j[:, 4 * c] + j[:, 4 * c + 1] + j[:, 4 * c + 2] + j[:, 4 * c + 3]
         for c in range(4)]
    t = [j[:, c] + j[:, 4 + c] + j[:, 8 + c] + j[:, 12 + c]
         for c in range(4)]
    card = (jnp.stack(p, axis=1) + jnp.stack(t, axis=1))
    score_ref[...] = jnp.mean(
        2.0 * inter / jnp.maximum(card, 1.0), axis=0, keepdims=True)


def kernel(output, target):
    o2 = output.reshape(8, 4, 512 * 512)
    t2 = target.reshape(8, 512 * 512)
    bins = _make_sc_kernel()(o2, t2)
    score = pl.pallas_call(
        _combine_body,
        grid=(1,),
        in_specs=[pl.BlockSpec((_NW, 8, 16, 16), lambda i: (0, 0, 0, 0))],
        out_specs=pl.BlockSpec((1, 4), lambda i: (0, 0)),
        out_shape=jax.ShapeDtypeStruct((1, 4), jnp.float32),
    )(bins)
    return score[0]
